# Initial kernel scaffold; baseline (speedup 1.0000x reference)
import jax
import jax.numpy as jnp
from jax.experimental import pallas as pl


def kernel(f0, f1, f2, f3, f4, f5, f6, f7, f8, f9, f10, f11, f12, f13, f14, f15, f16, f17, f18, f19, f20, f21, f22, f23, f24, f25):
    raise NotImplementedError("write your pallas kernel here")


# speedup vs baseline: 1.2712x; 1.2712x over previous
import functools

import jax
import jax.numpy as jnp
from jax import lax
from jax.experimental import pallas as pl
from jax.experimental.pallas import tpu as pltpu
from jax.experimental.pallas import tpu_sc as plsc

_NUM_FIELDS = 26
_BATCH = 4096
_MAX_SIZE = 1000
_NC = 2
_NS = 16
_NW = _NC * _NS
_RPW = _BATCH // _NW
_CROWS = 32
_NCHUNK = _RPW // _CROWS
_NBUF = 3
_L = 16


def _zero_buf(zb):
    zeros = jnp.zeros((_L,), jnp.float32)

    def row_body(r, carry):
        def col_body(j, c2):
            zb[r, pl.ds(j * 64, _L)] = zeros
            zb[r, pl.ds(j * 64 + 16, _L)] = zeros
            zb[r, pl.ds(j * 64 + 32, _L)] = zeros
            zb[r, pl.ds(j * 64 + 48, _L)] = zeros
            return c2

        lax.fori_loop(0, 15, col_body, 0)
        zb[r, pl.ds(960, _L)] = zeros
        zb[r, pl.ds(984, _L)] = zeros
        return carry

    lax.fori_loop(0, _CROWS, row_body, 0)


def _poke(zb, fidx, f, c, on):
    iota = lax.iota(jnp.int32, _L)
    val = jnp.full((_L,), 1.0 if on else 0.0, jnp.float32)
    for j in range(_CROWS // _L):
        rows = iota + (j * _L)
        cols = fidx[f, pl.ds(c * _CROWS + j * _L, _L)]
        plsc.store_scatter(zb, [rows, cols], val)


def _sc_body(*refs):
    ins = refs[:_NUM_FIELDS]
    outs = refs[_NUM_FIELDS:2 * _NUM_FIELDS]
    sc = refs[2 * _NUM_FIELDS:]
    zbufs = sc[:_NBUF]
    fidx = sc[_NBUF]
    isem = sc[_NBUF + 1]
    sems = sc[_NBUF + 2:_NBUF + 2 + _NBUF]

    wid = lax.axis_index("s") * _NC + lax.axis_index("c")
    base = wid * _RPW

    idx_copies = [
        pltpu.async_copy(ins[f].at[pl.ds(base, _RPW)], fidx.at[f], isem)
        for f in range(_NUM_FIELDS)
    ]

    for b in range(_NBUF):
        _zero_buf(zbufs[b])

    for cp in idx_copies:
        cp.wait()

    handles = [None] * _NBUF
    prev = [None] * _NBUF
    for g in range(_NUM_FIELDS * _NCHUNK):
        f, c = divmod(g, _NCHUNK)
        s = g % _NBUF
        zb = zbufs[s]
        if handles[s] is not None:
            handles[s].wait()
            pf, pc = prev[s]
            _poke(zb, fidx, pf, pc, False)
        _poke(zb, fidx, f, c, True)
        rb = base + c * _CROWS
        handles[s] = pltpu.async_copy(zb, outs[f].at[pl.ds(rb, _CROWS)], sems[s])
        prev[s] = (f, c)
    for s in range(_NBUF):
        if handles[s] is not None:
            handles[s].wait()


_onehot = functools.partial(
    pl.kernel,
    mesh=plsc.VectorSubcoreMesh(core_axis_name="c", subcore_axis_name="s"),
    compiler_params=pltpu.CompilerParams(
        use_tc_tiling_on_sc=False, needs_layout_passes=False),
    out_type=tuple(
        jax.ShapeDtypeStruct((_BATCH, _MAX_SIZE), jnp.float32)
        for _ in range(_NUM_FIELDS)
    ),
    scratch_types=(
        [pltpu.VMEM((_CROWS, _MAX_SIZE), jnp.float32) for _ in range(_NBUF)]
        + [pltpu.VMEM((_NUM_FIELDS, _RPW), jnp.int32)]
        + [pltpu.SemaphoreType.DMA] * (1 + _NBUF)
    ),
)(_sc_body)


def kernel(f0, f1, f2, f3, f4, f5, f6, f7, f8, f9, f10, f11, f12, f13,
           f14, f15, f16, f17, f18, f19, f20, f21, f22, f23, f24, f25):
    fields = (f0, f1, f2, f3, f4, f5, f6, f7, f8, f9, f10, f11, f12, f13,
              f14, f15, f16, f17, f18, f19, f20, f21, f22, f23, f24, f25)
    fields = tuple(jnp.asarray(f, dtype=jnp.int32) for f in fields)
    return _onehot(*fields)


# speedup vs baseline: 1.6942x; 1.3328x over previous
import functools

import jax
import jax.numpy as jnp
from jax import lax
from jax.experimental import pallas as pl
from jax.experimental.pallas import tpu as pltpu
from jax.experimental.pallas import tpu_sc as plsc

_NUM_FIELDS = 26
_BATCH = 4096
_MAX_SIZE = 1000
_NC = 2
_NS = 16
_NW = _NC * _NS
_RPW = _BATCH // _NW
_CROWS = 32
_NCHUNK = _RPW // _CROWS
_NBUF = 3
_L = 16

_N_SC = 0
_N_TC = _NUM_FIELDS - _N_SC
_TC_ROWS = 128


def _zero_buf(zb):
    zeros = jnp.zeros((_L,), jnp.float32)

    def row_body(r, carry):
        def col_body(j, c2):
            zb[r, pl.ds(j * 64, _L)] = zeros
            zb[r, pl.ds(j * 64 + 16, _L)] = zeros
            zb[r, pl.ds(j * 64 + 32, _L)] = zeros
            zb[r, pl.ds(j * 64 + 48, _L)] = zeros
            return c2

        lax.fori_loop(0, 15, col_body, 0)
        zb[r, pl.ds(960, _L)] = zeros
        zb[r, pl.ds(984, _L)] = zeros
        return carry

    lax.fori_loop(0, _CROWS, row_body, 0)


def _poke(zb, fidx, f, c, on):
    iota = lax.iota(jnp.int32, _L)
    val = jnp.full((_L,), 1.0 if on else 0.0, jnp.float32)
    for j in range(_CROWS // _L):
        rows = iota + (j * _L)
        cols = fidx[f, pl.ds(c * _CROWS + j * _L, _L)]
        plsc.store_scatter(zb, [rows, cols], val)


def _make_sc(n):
    def body(*refs):
        ins = refs[:n]
        outs = refs[n:2 * n]
        sc = refs[2 * n:]
        zbufs = sc[:_NBUF]
        fidx = sc[_NBUF]
        isem = sc[_NBUF + 1]
        sems = sc[_NBUF + 2:_NBUF + 2 + _NBUF]

        wid = lax.axis_index("s") * _NC + lax.axis_index("c")
        base = wid * _RPW

        idx_copies = [
            pltpu.async_copy(ins[f].at[pl.ds(base, _RPW)], fidx.at[f], isem)
            for f in range(n)
        ]

        for b in range(_NBUF):
            _zero_buf(zbufs[b])

        for cp in idx_copies:
            cp.wait()

        handles = [None] * _NBUF
        prev = [None] * _NBUF
        for g in range(n * _NCHUNK):
            f, c = divmod(g, _NCHUNK)
            s = g % _NBUF
            zb = zbufs[s]
            if handles[s] is not None:
                handles[s].wait()
                pf, pc = prev[s]
                _poke(zb, fidx, pf, pc, on=False)
            _poke(zb, fidx, f, c, on=True)
            rb = base + c * _CROWS
            handles[s] = pltpu.async_copy(
                zb, outs[f].at[pl.ds(rb, _CROWS)], sems[s])
            prev[s] = (f, c)
        for s in range(_NBUF):
            if handles[s] is not None:
                handles[s].wait()

    return pl.kernel(
        body,
        mesh=plsc.VectorSubcoreMesh(core_axis_name="c", subcore_axis_name="s"),
        compiler_params=pltpu.CompilerParams(
            use_tc_tiling_on_sc=False, needs_layout_passes=False),
        out_type=tuple(
            jax.ShapeDtypeStruct((_BATCH, _MAX_SIZE), jnp.float32)
            for _ in range(n)
        ),
        scratch_types=(
            [pltpu.VMEM((_CROWS, _MAX_SIZE), jnp.float32) for _ in range(_NBUF)]
            + [pltpu.VMEM((n, _RPW), jnp.int32)]
            + [pltpu.SemaphoreType.DMA] * (1 + _NBUF)
        ),
    )


_sc_call = _make_sc(_N_SC) if _N_SC else None


def _make_tc(n):
    def body(*refs):
        ins = refs[:n]
        outs = refs[n:]
        cols = lax.broadcasted_iota(jnp.int32, (_TC_ROWS, _MAX_SIZE), 1)
        for f in range(n):
            idx = ins[f][...]
            outs[f][...] = (cols == idx).astype(jnp.float32)

    return pl.pallas_call(
        body,
        grid=(_BATCH // _TC_ROWS,),
        in_specs=[pl.BlockSpec((_TC_ROWS, 1), lambda i: (i, 0))] * n,
        out_specs=[pl.BlockSpec((_TC_ROWS, _MAX_SIZE), lambda i: (i, 0))] * n,
        out_shape=[jax.ShapeDtypeStruct((_BATCH, _MAX_SIZE), jnp.float32)] * n,
    )


_tc_call = _make_tc(_N_TC) if _N_TC else None


def kernel(f0, f1, f2, f3, f4, f5, f6, f7, f8, f9, f10, f11, f12, f13,
           f14, f15, f16, f17, f18, f19, f20, f21, f22, f23, f24, f25):
    fields = (f0, f1, f2, f3, f4, f5, f6, f7, f8, f9, f10, f11, f12, f13,
              f14, f15, f16, f17, f18, f19, f20, f21, f22, f23, f24, f25)
    fields = tuple(jnp.asarray(f, dtype=jnp.int32) for f in fields)
    outs_sc = _sc_call(*fields[_N_TC:]) if _N_SC else ()
    if _N_TC:
        cols = tuple(f.reshape(_BATCH, 1) for f in fields[:_N_TC])
        outs_tc = tuple(_tc_call(*cols))
    else:
        outs_tc = ()
    return outs_tc + tuple(outs_sc)


# speedup vs baseline: 1.8982x; 1.1204x over previous
import functools

import jax
import jax.numpy as jnp
from jax import lax
from jax.experimental import pallas as pl
from jax.experimental.pallas import tpu as pltpu
from jax.experimental.pallas import tpu_sc as plsc

_NUM_FIELDS = 26
_BATCH = 4096
_MAX_SIZE = 1000
_NC = 2
_NS = 16
_NW = _NC * _NS
_RPW = _BATCH // _NW
_CROWS = 32
_NCHUNK = _RPW // _CROWS
_NBUF = 3
_L = 16

_N_SC = 0
_N_TC = _NUM_FIELDS - _N_SC
_TC_ROWS = 256


def _zero_buf(zb):
    zeros = jnp.zeros((_L,), jnp.float32)

    def row_body(r, carry):
        def col_body(j, c2):
            zb[r, pl.ds(j * 64, _L)] = zeros
            zb[r, pl.ds(j * 64 + 16, _L)] = zeros
            zb[r, pl.ds(j * 64 + 32, _L)] = zeros
            zb[r, pl.ds(j * 64 + 48, _L)] = zeros
            return c2

        lax.fori_loop(0, 15, col_body, 0)
        zb[r, pl.ds(960, _L)] = zeros
        zb[r, pl.ds(984, _L)] = zeros
        return carry

    lax.fori_loop(0, _CROWS, row_body, 0)


def _poke(zb, fidx, f, c, on):
    iota = lax.iota(jnp.int32, _L)
    val = jnp.full((_L,), 1.0 if on else 0.0, jnp.float32)
    for j in range(_CROWS // _L):
        rows = iota + (j * _L)
        cols = fidx[f, pl.ds(c * _CROWS + j * _L, _L)]
        plsc.store_scatter(zb, [rows, cols], val)


def _make_sc(n):
    def body(*refs):
        ins = refs[:n]
        outs = refs[n:2 * n]
        sc = refs[2 * n:]
        zbufs = sc[:_NBUF]
        fidx = sc[_NBUF]
        isem = sc[_NBUF + 1]
        sems = sc[_NBUF + 2:_NBUF + 2 + _NBUF]

        wid = lax.axis_index("s") * _NC + lax.axis_index("c")
        base = wid * _RPW

        idx_copies = [
            pltpu.async_copy(ins[f].at[pl.ds(base, _RPW)], fidx.at[f], isem)
            for f in range(n)
        ]

        for b in range(_NBUF):
            _zero_buf(zbufs[b])

        for cp in idx_copies:
            cp.wait()

        handles = [None] * _NBUF
        prev = [None] * _NBUF
        for g in range(n * _NCHUNK):
            f, c = divmod(g, _NCHUNK)
            s = g % _NBUF
            zb = zbufs[s]
            if handles[s] is not None:
                handles[s].wait()
                pf, pc = prev[s]
                _poke(zb, fidx, pf, pc, on=False)
            _poke(zb, fidx, f, c, on=True)
            rb = base + c * _CROWS
            handles[s] = pltpu.async_copy(
                zb, outs[f].at[pl.ds(rb, _CROWS)], sems[s])
            prev[s] = (f, c)
        for s in range(_NBUF):
            if handles[s] is not None:
                handles[s].wait()

    return pl.kernel(
        body,
        mesh=plsc.VectorSubcoreMesh(core_axis_name="c", subcore_axis_name="s"),
        compiler_params=pltpu.CompilerParams(
            use_tc_tiling_on_sc=False, needs_layout_passes=False),
        out_type=tuple(
            jax.ShapeDtypeStruct((_BATCH, _MAX_SIZE), jnp.float32)
            for _ in range(n)
        ),
        scratch_types=(
            [pltpu.VMEM((_CROWS, _MAX_SIZE), jnp.float32) for _ in range(_NBUF)]
            + [pltpu.VMEM((n, _RPW), jnp.int32)]
            + [pltpu.SemaphoreType.DMA] * (1 + _NBUF)
        ),
    )


_sc_call = _make_sc(_N_SC) if _N_SC else None


def _make_tc(n):
    def body(idx_ref, *outs):
        cols = lax.broadcasted_iota(jnp.int32, (_TC_ROWS, _MAX_SIZE), 1)
        idxs = idx_ref[...]
        for f in range(n):
            outs[f][...] = (cols == idxs[:, f:f + 1]).astype(jnp.float32)

    return pl.pallas_call(
        body,
        grid=(_BATCH // _TC_ROWS,),
        in_specs=[pl.BlockSpec((_TC_ROWS, n), lambda i: (i, 0))],
        out_specs=[pl.BlockSpec((_TC_ROWS, _MAX_SIZE), lambda i: (i, 0))] * n,
        out_shape=[jax.ShapeDtypeStruct((_BATCH, _MAX_SIZE), jnp.float32)] * n,
        compiler_params=pltpu.CompilerParams(
            vmem_limit_bytes=120 * 1024 * 1024),
    )


_tc_call = _make_tc(_N_TC) if _N_TC else None


def kernel(f0, f1, f2, f3, f4, f5, f6, f7, f8, f9, f10, f11, f12, f13,
           f14, f15, f16, f17, f18, f19, f20, f21, f22, f23, f24, f25):
    fields = (f0, f1, f2, f3, f4, f5, f6, f7, f8, f9, f10, f11, f12, f13,
              f14, f15, f16, f17, f18, f19, f20, f21, f22, f23, f24, f25)
    fields = tuple(jnp.asarray(f, dtype=jnp.int32) for f in fields)
    outs_sc = _sc_call(*fields[_N_TC:]) if _N_SC else ()
    if _N_TC:
        stacked = jnp.stack(fields[:_N_TC], axis=1)
        outs_tc = tuple(_tc_call(stacked))
    else:
        outs_tc = ()
    return outs_tc + tuple(outs_sc)
